# baseline (speedup 1.0000x reference)
import functools

import jax
import jax.numpy as jnp
from jax import lax
from jax.experimental import pallas as pl
from jax.experimental.pallas import tpu as pltpu
from jax.experimental.pallas import tpu_sc as plsc

N_ = 10000
F_ = 320000
C_ = 128
NC_ = 2
NS_ = 16
NW_ = NC_ * NS_
CHUNK_ = 128
NFULL_ = 78
NEXTRA_ = (F_ - NW_ * NFULL_ * CHUNK_) // CHUNK_
ROWS_W_ = NFULL_ * CHUNK_

BT_ = 2000
NB_ = F_ // BT_


def _sc_body(inp_hbm, sw_hbm, i0_hbm, i1_hbm, i2_hbm, v2f_hbm,
             iv0, iv1, iv2, swv,
             ra0, ra1, ra2, rb0, rb1, rb2, sa, sb):
  wid = lax.axis_index("s") * NC_ + lax.axis_index("c")
  wbase = wid * ROWS_W_

  pltpu.sync_copy(i0_hbm.at[pl.ds(wbase, ROWS_W_)], iv0)
  pltpu.sync_copy(i1_hbm.at[pl.ds(wbase, ROWS_W_)], iv1)
  pltpu.sync_copy(i2_hbm.at[pl.ds(wbase, ROWS_W_)], iv2)
  pltpu.sync_copy(sw_hbm, swv)

  nsl = C_ // 16
  w0 = [swv[0, pl.ds(s * 16, 16)] for s in range(nsl)]
  w1 = [swv[1, pl.ds(s * 16, 16)] for s in range(nsl)]
  w2 = [swv[2, pl.ds(s * 16, 16)] for s in range(nsl)]

  def issue(bufs, sem, j):
    off = j * CHUNK_
    pltpu.async_copy(inp_hbm.at[iv0.at[pl.ds(off, CHUNK_)]], bufs[0], sem)
    pltpu.async_copy(inp_hbm.at[iv1.at[pl.ds(off, CHUNK_)]], bufs[1], sem)
    pltpu.async_copy(inp_hbm.at[iv2.at[pl.ds(off, CHUNK_)]], bufs[2], sem)

  def drain(bufs, sem, j):
    off = j * CHUNK_
    pltpu.make_async_copy(inp_hbm.at[iv0.at[pl.ds(off, CHUNK_)]], bufs[0],
                          sem).wait()
    pltpu.make_async_copy(inp_hbm.at[iv1.at[pl.ds(off, CHUNK_)]], bufs[1],
                          sem).wait()
    pltpu.make_async_copy(inp_hbm.at[iv2.at[pl.ds(off, CHUNK_)]], bufs[2],
                          sem).wait()

  def combine(bufs):
    def row(r, carry):
      for s in range(nsl):
        sl = pl.ds(s * 16, 16)
        bufs[0][r, sl] = (bufs[0][r, sl] * w0[s] + bufs[1][r, sl] * w1[s]
                          + bufs[2][r, sl] * w2[s])
      return carry

    lax.fori_loop(0, CHUNK_, row, 0)

  def store(bufs, base):
    pltpu.sync_copy(bufs[0], v2f_hbm.at[pl.ds(base, CHUNK_)])

  bufs_a = (ra0, ra1, ra2)
  bufs_b = (rb0, rb1, rb2)

  issue(bufs_a, sa, 0)

  def body(i, carry):
    j0 = 2 * i
    issue(bufs_b, sb, j0 + 1)
    drain(bufs_a, sa, j0)
    combine(bufs_a)
    store(bufs_a, wbase + j0 * CHUNK_)

    @pl.when(j0 + 2 < NFULL_)
    def _():
      issue(bufs_a, sa, j0 + 2)

    drain(bufs_b, sb, j0 + 1)
    combine(bufs_b)
    store(bufs_b, wbase + (j0 + 1) * CHUNK_)
    return carry

  lax.fori_loop(0, NFULL_ // 2, body, 0)

  @pl.when(wid < NEXTRA_)
  def _():
    base = (NW_ * NFULL_ + wid) * CHUNK_
    pltpu.sync_copy(i0_hbm.at[pl.ds(base, CHUNK_)], iv0.at[pl.ds(0, CHUNK_)])
    pltpu.sync_copy(i1_hbm.at[pl.ds(base, CHUNK_)], iv1.at[pl.ds(0, CHUNK_)])
    pltpu.sync_copy(i2_hbm.at[pl.ds(base, CHUNK_)], iv2.at[pl.ds(0, CHUNK_)])
    issue(bufs_a, sa, 0)
    drain(bufs_a, sa, 0)
    combine(bufs_a)
    store(bufs_a, base)


@functools.lru_cache(maxsize=None)
def _get_sc_combine():
  return pl.kernel(
    out_type=jax.ShapeDtypeStruct((F_, C_), jnp.float32),
    mesh=plsc.VectorSubcoreMesh(core_axis_name="c", subcore_axis_name="s"),
    scratch_types=[
        pltpu.VMEM((ROWS_W_,), jnp.int32),
        pltpu.VMEM((ROWS_W_,), jnp.int32),
        pltpu.VMEM((ROWS_W_,), jnp.int32),
        pltpu.VMEM((8, C_), jnp.float32),
        pltpu.VMEM((CHUNK_, C_), jnp.float32),
        pltpu.VMEM((CHUNK_, C_), jnp.float32),
        pltpu.VMEM((CHUNK_, C_), jnp.float32),
        pltpu.VMEM((CHUNK_, C_), jnp.float32),
        pltpu.VMEM((CHUNK_, C_), jnp.float32),
        pltpu.VMEM((CHUNK_, C_), jnp.float32),
        pltpu.SemaphoreType.DMA,
        pltpu.SemaphoreType.DMA,
    ],
  )(_sc_body)


def _ab_body(v2f, dw, bb, gb, out, acc):
  p = pl.program_id(0)
  r = jnp.dot(v2f[...], dw[...], preferred_element_type=jnp.float32)
  r = jnp.maximum(r + bb[0, :][None, :], 0.0)

  @pl.when(p == 0)
  def _():
    s = jnp.sum(r, axis=0)
    s2 = jnp.sum(r * r, axis=0)
    upd = jnp.concatenate(
        [s[None, :], s2[None, :], jnp.zeros((6, C_), jnp.float32)], axis=0)

    @pl.when(pl.program_id(1) == 0)
    def _():
      acc[...] = upd

    @pl.when(pl.program_id(1) != 0)
    def _():
      acc[...] = acc[...] + upd

  @pl.when(p == 1)
  def _():
    mean = acc[0, :] / F_
    var = acc[1, :] / F_ - mean * mean
    inv = gb[0, :] / jnp.sqrt(var + 1e-5)
    out[...] = (r - mean[None, :]) * inv[None, :] + gb[1, :][None, :]


def kernel(inputs, face, spatial_weights, depth_weights, biases,
           bn_gamma, bn_beta):
  face32 = face.astype(jnp.int32)
  ft = face32.T
  i0 = ft[0]
  i1 = ft[1]
  i2 = ft[2]

  sw8 = jnp.pad(spatial_weights[:, :, 0], ((0, 5), (0, 0)))
  bb8 = jnp.pad(biases, ((0, 7), (0, 0)))
  gb8 = jnp.pad(jnp.stack([bn_gamma, bn_beta]), ((0, 6), (0, 0)))

  v2f = _get_sc_combine()(inputs, sw8, i0, i1, i2)

  out = pl.pallas_call(
      _ab_body,
      grid=(2, NB_),
      in_specs=[
          pl.BlockSpec((BT_, C_), lambda p, i: (i, 0)),
          pl.BlockSpec((C_, C_), lambda p, i: (0, 0)),
          pl.BlockSpec((8, C_), lambda p, i: (0, 0)),
          pl.BlockSpec((8, C_), lambda p, i: (0, 0)),
      ],
      out_specs=pl.BlockSpec((BT_, C_),
                             lambda p, i: (jnp.where(p == 1, i, 0), 0)),
      out_shape=jax.ShapeDtypeStruct((F_, C_), jnp.float32),
      scratch_shapes=[pltpu.VMEM((8, C_), jnp.float32)],
  )(v2f, depth_weights, bb8, gb8)

  return out
